# baseline (speedup 1.0000x reference)
import jax
import jax.numpy as jnp
from jax import lax
from jax.experimental import pallas as pl
from jax.experimental.pallas import tpu as pltpu
from jax.experimental.pallas import tpu_sc as plsc

_ND = 8
_NT = 4
_DW = 128
_TW = 512
_TH = 256
_TQ = 128


def _sc_body(q_hbm, pos_hbm, out_hbm, stage, pos_t, qb):
    bsz = q_hbm.shape[0]
    wid = lax.axis_index("s") * 2 + lax.axis_index("c")
    d0 = (wid % _ND) * _DW
    tbase = (wid // _ND) * _TW
    for th in range(_TW // _TH):
        t0 = tbase + th * _TH
        for k in range(_TH // _TQ):
            pltpu.sync_copy(
                pos_hbm.at[pl.ds(t0 + k * _TQ, _TQ), pl.ds(d0, _DW)], stage)

            def tr_t(t1, _):
                tvec = jnp.full((16,), k * _TQ + t1, jnp.int32)

                def tr_d(dv, _):
                    dvec = dv * 16 + lax.iota(jnp.int32, 16)
                    v = stage[t1, pl.ds(dv * 16, 16)]
                    plsc.store_scatter(pos_t, [dvec, tvec], v)
                    return 0

                lax.fori_loop(0, _DW // 16, tr_d, 0)
                return 0

            lax.fori_loop(0, _TQ, tr_t, 0)
        for b in range(bsz):
            pltpu.sync_copy(q_hbm.at[b, pl.ds(d0, _DW), pl.ds(t0, _TH)], qb)

            def add_d(d1, _):
                def add_t(tv, _):
                    sl = pl.ds(tv * 16, 16)
                    qb[d1, sl] = qb[d1, sl] + pos_t[d1, sl]
                    return 0

                lax.fori_loop(0, _TH // 16, add_t, 0)
                return 0

            lax.fori_loop(0, _DW, add_d, 0)
            pltpu.sync_copy(qb, out_hbm.at[b, pl.ds(d0, _DW), pl.ds(t0, _TH)])


def kernel(q, pos_embed):
    bsz, d_model, q_frm = q.shape
    mesh = plsc.VectorSubcoreMesh(core_axis_name="c", subcore_axis_name="s")
    f = pl.kernel(
        _sc_body,
        mesh=mesh,
        out_type=jax.ShapeDtypeStruct((bsz, d_model, q_frm), q.dtype),
        scratch_types=[
            pltpu.VMEM((_TQ, _DW), jnp.float32),
            pltpu.VMEM((_DW, _TH), jnp.float32),
            pltpu.VMEM((_DW, _TH), jnp.float32),
        ],
        compiler_params=pltpu.CompilerParams(
            use_tc_tiling_on_sc=False, needs_layout_passes=False),
    )
    return f(q, pos_embed)
